# baseline (speedup 1.0000x reference)
import functools

import jax
import jax.numpy as jnp
from jax.experimental import pallas as pl
from jax.experimental.pallas import tpu as pltpu

N = 134400
C = 64
W = 16
RT = 16384
NB = (N + RT - 1) // RT


def _body(stu_ref, tea_ref, siou_ref, tiou_ref, sgt_ref, tgt_ref, ms_ref,
          out_ref, acc_ref):
    pid = pl.program_id(0)

    @pl.when(pid == 0)
    def _init():
        acc_ref[0] = 0.0
        acc_ref[1] = 0.0

    nvalid = N - pid * RT

    lane1 = jax.lax.broadcasted_iota(jnp.int32, (RT,), 0)
    m = jnp.logical_and(tiou_ref[...] >= siou_ref[...],
                        tgt_ref[...] == sgt_ref[...])
    m = jnp.logical_and(m, ms_ref[...])
    m = jnp.logical_and(m, lane1 < nvalid)
    mf = m.astype(jnp.float32)

    t = tea_ref[...]
    s = stu_ref[...]

    a8 = jax.lax.broadcasted_iota(jnp.int32, (8, C), 0) & 3
    c8 = jax.lax.broadcasted_iota(jnp.int32, (8, C), 1) // W
    p8 = (a8 == c8).astype(jnp.bfloat16)

    tb = t.astype(jnp.bfloat16)
    sb = s.astype(jnp.bfloat16)
    et = jnp.exp(tb)
    es = jnp.exp(sb)
    z = et * (tb - sb)
    f32 = jnp.float32
    bt8 = jax.lax.dot(p8, et, preferred_element_type=f32)
    bs8 = jax.lax.dot(p8, es, preferred_element_type=f32)
    zg8 = jax.lax.dot(p8, z, preferred_element_type=f32)

    l4 = jnp.log(bt8[0:4, :] / bs8[0:4, :])
    g4 = zg8[0:4, :] / bt8[0:4, :]

    row_kl = jnp.sum(g4 - l4, axis=0)
    row_kl = jnp.where(lane1 < nvalid, row_kl, 0.0)

    acc_ref[0] += jnp.sum(row_kl * mf)
    acc_ref[1] += jnp.sum(mf)

    @pl.when(pid == NB - 1)
    def _fin():
        out_ref[0, 0] = acc_ref[0] / (jnp.maximum(acc_ref[1], 1.0) * C)


@functools.partial(jax.jit, static_argnames=())
def kernel(stu_distri, tea_distri, stu_candidate_iou, tea_candidate_iou,
           stu_target_gt_idx, tea_target_gt_idx, Ms):
    st = stu_distri.T
    tt = tea_distri.T
    sgt = stu_target_gt_idx
    tgt = tea_target_gt_idx
    if sgt.dtype != jnp.int32:
        sgt = sgt.astype(jnp.int32)
        tgt = tgt.astype(jnp.int32)

    row_spec = pl.BlockSpec((C, RT), lambda i: (0, i))
    vec_spec = pl.BlockSpec((RT,), lambda i: (i,))

    out = pl.pallas_call(
        _body,
        grid=(NB,),
        in_specs=[row_spec, row_spec] + [vec_spec] * 5,
        out_specs=pl.BlockSpec(memory_space=pltpu.SMEM),
        out_shape=jax.ShapeDtypeStruct((1, 1), jnp.float32),
        scratch_shapes=[pltpu.SMEM((2,), jnp.float32)],
        compiler_params=pltpu.CompilerParams(
            dimension_semantics=("arbitrary",)),
    )(st, tt, stu_candidate_iou, tea_candidate_iou, sgt, tgt, Ms)
    return out[0, 0]
